# baseline (speedup 1.0000x reference)
import functools

import jax
import jax.numpy as jnp
from jax import lax
from jax.experimental import pallas as pl
from jax.experimental.pallas import tpu as pltpu
from jax.experimental.pallas import tpu_sc as plsc

N = 10000
E = 320000
IN = 128
HC = 64
TW = 2 * HC
EPS = 1e-5

NC, NS = 2, 16
NW = NC * NS
EPW = E // NW
GK = 400
SK = 3200
LANES = 16

_MESH = plsc.VectorSubcoreMesh(
    core_axis_name="c", subcore_axis_name="s", num_cores=NC, num_subcores=NS)

_BN_S = (1.0 + EPS) ** -0.5


def _tc_pre1_body(x_ref, wd_ref, wr_ref, ba_ref, t_out):
    xb = x_ref[...]
    bmat = lax.dot_general(xb, wr_ref[...], (((1,), (1,)), ((), ())),
                           preferred_element_type=jnp.float32)
    c = lax.dot_general(xb, wd_ref[...], (((1,), (1,)), ((), ())),
                        preferred_element_type=jnp.float32) \
        + ba_ref[...][None, :]
    t_out[...] = jnp.concatenate([c, bmat], axis=1)


def _tc_pre1(x, wd, wr, ba):
    return pl.pallas_call(
        _tc_pre1_body,
        out_shape=jax.ShapeDtypeStruct((N, TW), jnp.float32),
    )(x, wd, wr, ba)


def _tc_pre_body(agg_ref, g_ref, be_ref, wd_ref, wr_ref, ba_ref, t_out):
    a = jnp.max(agg_ref[...], axis=0)
    a = jnp.where(jnp.isfinite(a), a, 0.0)
    s = g_ref[...] * _BN_S
    h = jnp.maximum(a * s[:, None] + be_ref[...][:, None], 0.0)
    bmat = lax.dot_general(h, wr_ref[...], (((0,), (1,)), ((), ())),
                           preferred_element_type=jnp.float32)
    c = lax.dot_general(h, wd_ref[...], (((0,), (1,)), ((), ())),
                        preferred_element_type=jnp.float32) \
        + ba_ref[...][None, :]
    t_out[...] = jnp.concatenate([c, bmat], axis=1)


def _tc_pre(agg_t, g, be, wd, wr, ba):
    return pl.pallas_call(
        _tc_pre_body,
        out_shape=jax.ShapeDtypeStruct((N, TW), jnp.float32),
    )(agg_t, g, be, wd, wr, ba)


_EB = 6400


def _tc_edge_body(h_ref, w_ref, b_ref, o_ref):
    hb = h_ref[...]
    m = lax.dot_general(w_ref[...], hb, (((1,), (1,)), ((), ())),
                        preferred_element_type=jnp.float32)
    o_ref[...] = m + b_ref[...][:, None]


def _tc_edge(h, wb_pad, bb):
    grid = E // _EB
    return pl.pallas_call(
        _tc_edge_body,
        grid=(grid,),
        in_specs=[
            pl.BlockSpec((_EB, TW), lambda i: (i, 0)),
            pl.BlockSpec((HC, TW), lambda i: (0, 0)),
            pl.BlockSpec((HC,), lambda i: (0,)),
        ],
        out_specs=pl.BlockSpec((HC, _EB), lambda i: (0, i)),
        out_shape=jax.ShapeDtypeStruct((HC, E), jnp.float32),
    )(h, wb_pad, bb)


def _tc_head_body(agg_ref, g_ref, be_ref, w1_ref, b1_ref, w2_ref, b2_ref,
                  w3_ref, b3_ref, w4_ref, b4_ref, o_ref):
    a = jnp.max(agg_ref[...], axis=0)
    a = jnp.where(jnp.isfinite(a), a, 0.0)
    s = g_ref[...] * _BN_S
    h = jnp.maximum(a * s[:, None] + be_ref[...][:, None], 0.0)
    h = jnp.maximum(lax.dot_general(w1_ref[...], h, (((1,), (0,)), ((), ())),
                                    preferred_element_type=jnp.float32)
                    + b1_ref[...][:, None], 0.0)
    h = jnp.maximum(lax.dot_general(w2_ref[...], h, (((1,), (0,)), ((), ())),
                                    preferred_element_type=jnp.float32)
                    + b2_ref[...][:, None], 0.0)
    h = jnp.maximum(lax.dot_general(w3_ref[...], h, (((1,), (0,)), ((), ())),
                                    preferred_element_type=jnp.float32)
                    + b3_ref[...][:, None], 0.0)
    o_ref[...] = lax.dot_general(h, w4_ref[...], (((0,), (1,)), ((), ())),
                                 preferred_element_type=jnp.float32) \
        + b4_ref[...][None, :]


def _tc_head(agg_t, g, be, w1, b1, w2, b2, w3, b3, w4, b4):
    return pl.pallas_call(
        _tc_head_body,
        out_shape=jax.ShapeDtypeStruct((N, w4.shape[0]), jnp.float32),
    )(agg_t, g, be, w1, b1, w2, b2, w3, b3, w4, b4)


def _sc_gather_body(src_hbm, dst_hbm, t_hbm, out_hbm,
                    idxd, idxs, bufd, bufs, sem1, sem2):
    wid = lax.axis_index("s") * NC + lax.axis_index("c")
    base_w = wid * EPW

    @pl.loop(0, EPW // GK)
    def _chunk(j):
        base = pl.multiple_of(base_w + j * GK, 8)
        pltpu.sync_copy(dst_hbm.at[pl.ds(base, GK)], idxd)
        pltpu.sync_copy(src_hbm.at[pl.ds(base, GK)], idxs)
        cp1 = pltpu.async_copy(t_hbm.at[idxd], bufd, sem1)
        cp2 = pltpu.async_copy(t_hbm.at[idxs], bufs, sem2)
        cp1.wait()
        cp2.wait()

        @pl.loop(0, GK)
        def _row(r):
            for c in range(HC // LANES):
                sl = pl.ds(c * LANES, LANES)
                sr = pl.ds(HC + c * LANES, LANES)
                bufd[r, sl] = jnp.maximum(bufd[r, sl] + bufs[r, sr], 0.0)

        pltpu.sync_copy(bufd, out_hbm.at[pl.ds(base, GK), :])


def _sc_gather(src, dst, t_tab):
    f = functools.partial(
        pl.kernel,
        mesh=_MESH,
        compiler_params=pltpu.CompilerParams(needs_layout_passes=False),
        out_type=jax.ShapeDtypeStruct((E, TW), jnp.float32),
        scratch_types=[
            pltpu.VMEM((GK,), jnp.int32),
            pltpu.VMEM((GK,), jnp.int32),
            pltpu.VMEM((GK, TW), jnp.float32),
            pltpu.VMEM((GK, TW), jnp.float32),
            pltpu.SemaphoreType.DMA,
            pltpu.SemaphoreType.DMA,
        ],
    )(_sc_gather_body)
    return f(src, dst, t_tab)


NFP = 8
NEP = 4
EPQ = E // NEP


def _sc_scatmax_body(dst_hbm, mt_hbm, out_hbm, idxb, vals, scr, flags, *accs):
    wid = lax.axis_index("s") * NC + lax.axis_index("c")
    p = wid // NFP
    f0 = pl.multiple_of((wid % NFP) * NFP, 8)
    base_e = p * EPQ
    neg = jnp.full((LANES,), -jnp.inf, jnp.float32)

    @pl.loop(0, N // LANES)
    def _init(i):
        sl = pl.ds(i * LANES, LANES)
        for acc in accs:
            acc[sl] = neg

    @pl.loop(0, EPQ // SK)
    def _chunk(j):
        e0 = pl.multiple_of(base_e + j * SK, 128)
        pltpu.sync_copy(dst_hbm.at[pl.ds(e0, SK)], idxb)
        pltpu.sync_copy(mt_hbm.at[pl.ds(f0, NFP), pl.ds(e0, SK)], vals)

        @pl.loop(0, SK // LANES)
        def _probe(g):
            sl = pl.ds(g * LANES, LANES)
            idxv = idxb[sl]
            lanes = lax.iota(jnp.int32, LANES).astype(jnp.float32)
            plsc.store_scatter(scr, [idxv], lanes)
            rd = plsc.load_gather(scr, [idxv])
            flags[sl] = plsc.all_reduce_population_count(rd != lanes)

        @pl.loop(0, SK // LANES)
        def _grp(g):
            sl = pl.ds(g * LANES, LANES)
            idxv = idxb[sl]
            for f, acc in enumerate(accs):
                v = vals[f, sl]
                cur = plsc.load_gather(acc, [idxv])
                plsc.store_scatter(acc, [idxv], jnp.maximum(v, cur))

            @pl.when(flags[sl][0] > 0)
            def _repair():
                for f, acc in enumerate(accs):
                    v = vals[f, sl]
                    cur = plsc.load_gather(acc, [idxv])
                    m = jnp.maximum(v, cur)
                    back = plsc.load_gather(acc, [idxv])
                    cnt = jnp.sum((back < m).astype(jnp.int32))

                    def _cond(c):
                        return c > 0

                    def _body(c):
                        b1 = plsc.load_gather(acc, [idxv])
                        msk = b1 < m
                        plsc.store_scatter(acc, [idxv], m, mask=msk)
                        b2 = plsc.load_gather(acc, [idxv])
                        return jnp.sum((b2 < m).astype(jnp.int32))

                    lax.while_loop(_cond, _body, cnt)

    for f, acc in enumerate(accs):
        base = pl.multiple_of((p * HC + f0 + f) * N, 8)
        pltpu.sync_copy(acc, out_hbm.at[pl.ds(base, N)])


def _sc_scatmax(dst, m_t):
    f = functools.partial(
        pl.kernel,
        mesh=_MESH,
        compiler_params=pltpu.CompilerParams(needs_layout_passes=False),
        out_type=jax.ShapeDtypeStruct((NEP * HC * N,), jnp.float32),
        scratch_types=[
            pltpu.VMEM((SK,), jnp.int32),
            pltpu.VMEM((NFP, SK), jnp.float32),
            pltpu.VMEM((N,), jnp.float32),
            pltpu.VMEM((SK,), jnp.int32),
        ] + [pltpu.VMEM((N,), jnp.float32)] * NFP,
    )(_sc_scatmax_body)
    return jnp.reshape(f(dst, m_t), (NEP, HC, N))


def _pad_w(wb):
    return jnp.concatenate([wb, jnp.zeros_like(wb)], axis=1)


def kernel(x, edge_index, W1a, b1a, W1b, b1b, W2a, b2a, W2b, b2b, W3a, b3a,
           W3b, b3b, g1, be1, g2, be2, g3, be3, L1w, L1b, L2w, L2b, L3w, L3b,
           L4w, L4b):
    src = edge_index[0]
    dst = edge_index[1]

    t_tab = _tc_pre1(x, W1a[:, :IN] - W1a[:, IN:], W1a[:, IN:], b1a)
    h_e = _sc_gather(src, dst, t_tab)
    m_t = _tc_edge(h_e, _pad_w(W1b), b1b)
    agg = _sc_scatmax(dst, m_t)

    t_tab = _tc_pre(agg, g1, be1, W2a[:, :HC] - W2a[:, HC:], W2a[:, HC:], b2a)
    h_e = _sc_gather(src, dst, t_tab)
    m_t = _tc_edge(h_e, _pad_w(W2b), b2b)
    agg = _sc_scatmax(dst, m_t)

    t_tab = _tc_pre(agg, g2, be2, W3a[:, :HC] - W3a[:, HC:], W3a[:, HC:], b3a)
    h_e = _sc_gather(src, dst, t_tab)
    m_t = _tc_edge(h_e, _pad_w(W3b), b3b)
    agg = _sc_scatmax(dst, m_t)

    return _tc_head(agg, g3, be3, L1w, L1b, L2w, L2b, L3w, L3b, L4w, L4b)


# speedup vs baseline: 1.0079x; 1.0079x over previous
import functools

import jax
import jax.numpy as jnp
from jax import lax
from jax.experimental import pallas as pl
from jax.experimental.pallas import tpu as pltpu
from jax.experimental.pallas import tpu_sc as plsc

N = 10000
E = 320000
IN = 128
HC = 64
TW = 2 * HC
EPS = 1e-5

NC, NS = 2, 16
NW = NC * NS
EPW = E // NW
GK = 400
SK = 3200
LANES = 16

_MESH = plsc.VectorSubcoreMesh(
    core_axis_name="c", subcore_axis_name="s", num_cores=NC, num_subcores=NS)

_BN_S = (1.0 + EPS) ** -0.5


def _tc_pre1_body(x_ref, wd_ref, wr_ref, ba_ref, t_out):
    xb = x_ref[...]
    bmat = lax.dot_general(xb, wr_ref[...], (((1,), (1,)), ((), ())),
                           preferred_element_type=jnp.float32)
    c = lax.dot_general(xb, wd_ref[...], (((1,), (1,)), ((), ())),
                        preferred_element_type=jnp.float32) \
        + ba_ref[...][None, :]
    t_out[...] = jnp.concatenate([c, bmat], axis=1)


def _tc_pre1(x, wd, wr, ba):
    return pl.pallas_call(
        _tc_pre1_body,
        out_shape=jax.ShapeDtypeStruct((N, TW), jnp.float32),
    )(x, wd, wr, ba)


def _tc_pre_body(agg_ref, g_ref, be_ref, wd_ref, wr_ref, ba_ref, t_out):
    a = jnp.max(agg_ref[...], axis=0)
    a = jnp.where(jnp.isfinite(a), a, 0.0)
    s = g_ref[...] * _BN_S
    h = jnp.maximum(a * s[:, None] + be_ref[...][:, None], 0.0)
    bmat = lax.dot_general(h, wr_ref[...], (((0,), (1,)), ((), ())),
                           preferred_element_type=jnp.float32)
    c = lax.dot_general(h, wd_ref[...], (((0,), (1,)), ((), ())),
                        preferred_element_type=jnp.float32) \
        + ba_ref[...][None, :]
    t_out[...] = jnp.concatenate([c, bmat], axis=1)


def _tc_pre(agg_t, g, be, wd, wr, ba):
    return pl.pallas_call(
        _tc_pre_body,
        out_shape=jax.ShapeDtypeStruct((N, TW), jnp.float32),
    )(agg_t, g, be, wd, wr, ba)


_EB = 3200


def _tc_edge_body(h_ref, w_ref, b_ref, o_ref):
    hb = h_ref[...]
    m = lax.dot_general(w_ref[...], hb, (((1,), (1,)), ((), ())),
                        preferred_element_type=jnp.float32)
    o_ref[...] = m + b_ref[...][:, None]


def _tc_edge(h, wb_pad, bb2):
    grid = (E // 2) // _EB
    return pl.pallas_call(
        _tc_edge_body,
        grid=(grid,),
        in_specs=[
            pl.BlockSpec((_EB, 2 * TW), lambda i: (i, 0)),
            pl.BlockSpec((TW, 2 * TW), lambda i: (0, 0)),
            pl.BlockSpec((TW,), lambda i: (0,)),
        ],
        out_specs=pl.BlockSpec((TW, _EB), lambda i: (0, i)),
        out_shape=jax.ShapeDtypeStruct((TW, E // 2), jnp.float32),
    )(h, wb_pad, bb2)


def _tc_head_body(agg_ref, g_ref, be_ref, w1_ref, b1_ref, w2_ref, b2_ref,
                  w3_ref, b3_ref, w4_ref, b4_ref, o_ref):
    a = jnp.max(agg_ref[...], axis=0)
    a = jnp.where(jnp.isfinite(a), a, 0.0)
    s = g_ref[...] * _BN_S
    h = jnp.maximum(a * s[:, None] + be_ref[...][:, None], 0.0)
    h = jnp.maximum(lax.dot_general(w1_ref[...], h, (((1,), (0,)), ((), ())),
                                    preferred_element_type=jnp.float32)
                    + b1_ref[...][:, None], 0.0)
    h = jnp.maximum(lax.dot_general(w2_ref[...], h, (((1,), (0,)), ((), ())),
                                    preferred_element_type=jnp.float32)
                    + b2_ref[...][:, None], 0.0)
    h = jnp.maximum(lax.dot_general(w3_ref[...], h, (((1,), (0,)), ((), ())),
                                    preferred_element_type=jnp.float32)
                    + b3_ref[...][:, None], 0.0)
    o_ref[...] = lax.dot_general(h, w4_ref[...], (((0,), (1,)), ((), ())),
                                 preferred_element_type=jnp.float32) \
        + b4_ref[...][None, :]


def _tc_head(agg_t, g, be, w1, b1, w2, b2, w3, b3, w4, b4):
    return pl.pallas_call(
        _tc_head_body,
        out_shape=jax.ShapeDtypeStruct((N, w4.shape[0]), jnp.float32),
    )(agg_t, g, be, w1, b1, w2, b2, w3, b3, w4, b4)


GH = GK // 2


def _sc_gather_body(src_hbm, dst_hbm, t_hbm, out_hbm,
                    idxd, idxs, bufdl, bufdr, bufsl, bufsr, sem1, sem2):
    wid = lax.axis_index("s") * NC + lax.axis_index("c")
    base_w = wid * EPW

    @pl.loop(0, EPW // GK)
    def _chunk(j):
        base = pl.multiple_of(base_w + j * GK, 8)
        pltpu.sync_copy(dst_hbm.at[pl.ds(base, GK)], idxd)
        pltpu.sync_copy(src_hbm.at[pl.ds(base, GK)], idxs)
        cp1 = pltpu.async_copy(t_hbm.at[idxd.at[pl.ds(0, GH)]], bufdl, sem1)
        cp2 = pltpu.async_copy(t_hbm.at[idxd.at[pl.ds(GH, GH)]], bufdr, sem2)
        cp3 = pltpu.async_copy(t_hbm.at[idxs.at[pl.ds(0, GH)]], bufsl, sem1)
        cp4 = pltpu.async_copy(t_hbm.at[idxs.at[pl.ds(GH, GH)]], bufsr, sem2)
        cp1.wait()
        cp2.wait()
        cp3.wait()
        cp4.wait()

        @pl.loop(0, GH)
        def _row(r):
            for c in range(HC // LANES):
                sl = pl.ds(c * LANES, LANES)
                sr = pl.ds(HC + c * LANES, LANES)
                bufdl[r, sl] = jnp.maximum(bufdl[r, sl] + bufsl[r, sr], 0.0)
                bufdr[r, sl] = jnp.maximum(bufdr[r, sl] + bufsr[r, sr], 0.0)

        baseh = pl.multiple_of((base_w + j * GK) // 2, 8)
        pltpu.sync_copy(bufdl, out_hbm.at[pl.ds(baseh, GH), pl.ds(0, TW)])
        pltpu.sync_copy(bufdr, out_hbm.at[pl.ds(baseh, GH), pl.ds(TW, TW)])


def _sc_gather(src, dst, t_tab):
    f = functools.partial(
        pl.kernel,
        mesh=_MESH,
        compiler_params=pltpu.CompilerParams(needs_layout_passes=False),
        out_type=jax.ShapeDtypeStruct((E // 2, 2 * TW), jnp.float32),
        scratch_types=[
            pltpu.VMEM((GK,), jnp.int32),
            pltpu.VMEM((GK,), jnp.int32),
            pltpu.VMEM((GH, TW), jnp.float32),
            pltpu.VMEM((GH, TW), jnp.float32),
            pltpu.VMEM((GH, TW), jnp.float32),
            pltpu.VMEM((GH, TW), jnp.float32),
            pltpu.SemaphoreType.DMA,
            pltpu.SemaphoreType.DMA,
        ],
    )(_sc_gather_body)
    return f(src, dst, t_tab)


NFP = 8
NRG = TW // NFP
NCP = NW // NRG
EPQ = (E // 2) // NCP


def _sc_scatmax_body(dst_hbm, mt_hbm, out_hbm, idxb, vals, scr, flags, *accs):
    wid = lax.axis_index("s") * NC + lax.axis_index("c")
    p = wid // NRG
    rg = wid % NRG
    f0 = pl.multiple_of(rg * NFP, 8)
    base_e = rg // NFP * (E // 2) + p * EPQ
    base_c = p * EPQ
    neg = jnp.full((LANES,), -jnp.inf, jnp.float32)

    @pl.loop(0, N // LANES)
    def _init(i):
        sl = pl.ds(i * LANES, LANES)
        for acc in accs:
            acc[sl] = neg

    @pl.loop(0, EPQ // SK)
    def _chunk(j):
        e0 = pl.multiple_of(base_e + j * SK, 128)
        c0 = pl.multiple_of(base_c + j * SK, 128)
        pltpu.sync_copy(dst_hbm.at[pl.ds(e0, SK)], idxb)
        pltpu.sync_copy(mt_hbm.at[pl.ds(f0, NFP), pl.ds(c0, SK)], vals)

        @pl.loop(0, SK // LANES)
        def _probe(g):
            sl = pl.ds(g * LANES, LANES)
            idxv = idxb[sl]
            lanes = lax.iota(jnp.int32, LANES).astype(jnp.float32)
            plsc.store_scatter(scr, [idxv], lanes)
            rd = plsc.load_gather(scr, [idxv])
            flags[sl] = plsc.all_reduce_population_count(rd != lanes)

        @pl.loop(0, SK // LANES)
        def _grp(g):
            sl = pl.ds(g * LANES, LANES)
            idxv = idxb[sl]
            for f, acc in enumerate(accs):
                v = vals[f, sl]
                cur = plsc.load_gather(acc, [idxv])
                plsc.store_scatter(acc, [idxv], jnp.maximum(v, cur))

            @pl.when(flags[sl][0] > 0)
            def _repair():
                for f, acc in enumerate(accs):
                    v = vals[f, sl]
                    cur = plsc.load_gather(acc, [idxv])
                    m = jnp.maximum(v, cur)
                    back = plsc.load_gather(acc, [idxv])
                    cnt = jnp.sum((back < m).astype(jnp.int32))

                    def _cond(c):
                        return c > 0

                    def _body(c):
                        b1 = plsc.load_gather(acc, [idxv])
                        msk = b1 < m
                        plsc.store_scatter(acc, [idxv], m, mask=msk)
                        b2 = plsc.load_gather(acc, [idxv])
                        return jnp.sum((b2 < m).astype(jnp.int32))

                    lax.while_loop(_cond, _body, cnt)

    for f, acc in enumerate(accs):
        base = pl.multiple_of((p * TW + f0 + f) * N, 8)
        pltpu.sync_copy(acc, out_hbm.at[pl.ds(base, N)])


def _sc_scatmax(dst, m_t):
    f = functools.partial(
        pl.kernel,
        mesh=_MESH,
        compiler_params=pltpu.CompilerParams(needs_layout_passes=False),
        out_type=jax.ShapeDtypeStruct((NCP * TW * N,), jnp.float32),
        scratch_types=[
            pltpu.VMEM((SK,), jnp.int32),
            pltpu.VMEM((NFP, SK), jnp.float32),
            pltpu.VMEM((N,), jnp.float32),
            pltpu.VMEM((SK,), jnp.int32),
        ] + [pltpu.VMEM((N,), jnp.float32)] * NFP,
    )(_sc_scatmax_body)
    return jnp.reshape(f(dst, m_t), (NCP * 2, HC, N))


def _pad_w(wb):
    z = jnp.zeros_like(wb)
    top = jnp.concatenate([wb, z, z, z], axis=1)
    bot = jnp.concatenate([z, z, wb, z], axis=1)
    return jnp.concatenate([top, bot], axis=0)


def kernel(x, edge_index, W1a, b1a, W1b, b1b, W2a, b2a, W2b, b2b, W3a, b3a,
           W3b, b3b, g1, be1, g2, be2, g3, be3, L1w, L1b, L2w, L2b, L3w, L3b,
           L4w, L4b):
    src = edge_index[0]
    dst = edge_index[1]
    dst_lr = jnp.transpose(jnp.reshape(dst, (E // GK, 2, GK // 2)),
                           (1, 0, 2)).reshape(E)

    t_tab = _tc_pre1(x, W1a[:, :IN] - W1a[:, IN:], W1a[:, IN:], b1a)
    h_e = _sc_gather(src, dst, t_tab)
    m_t = _tc_edge(h_e, _pad_w(W1b), jnp.concatenate([b1b, b1b]))
    agg = _sc_scatmax(dst_lr, m_t)

    t_tab = _tc_pre(agg, g1, be1, W2a[:, :HC] - W2a[:, HC:], W2a[:, HC:], b2a)
    h_e = _sc_gather(src, dst, t_tab)
    m_t = _tc_edge(h_e, _pad_w(W2b), jnp.concatenate([b2b, b2b]))
    agg = _sc_scatmax(dst_lr, m_t)

    t_tab = _tc_pre(agg, g2, be2, W3a[:, :HC] - W3a[:, HC:], W3a[:, HC:], b3a)
    h_e = _sc_gather(src, dst, t_tab)
    m_t = _tc_edge(h_e, _pad_w(W3b), jnp.concatenate([b3b, b3b]))
    agg = _sc_scatmax(dst_lr, m_t)

    return _tc_head(agg, g3, be3, L1w, L1b, L2w, L2b, L3w, L3b, L4w, L4b)
